# Initial kernel scaffold; baseline (speedup 1.0000x reference)
import jax
import jax.numpy as jnp
from jax.experimental import pallas as pl


def kernel(x, edge_index, W1, b1, W2, b2, W3, b3, W4, b4, Wb, bb):
    raise NotImplementedError("write your pallas kernel here")


# speedup vs baseline: 21.8645x; 21.8645x over previous
import functools

import jax
import jax.numpy as jnp
from jax import lax
from jax.experimental import pallas as pl
from jax.experimental.pallas import tpu as pltpu
from jax.experimental.pallas import tpu_sc as plsc

N_NODES = 10000
NP = 10240
E = 320000
EP = 323584
D = 64
NC, NS = 2, 16
NW = NC * NS
CH = 128
NCH = (EP // NW) // CH
RPS = NP // NS
GC, GB = 4, 80

_MESH = plsc.VectorSubcoreMesh(core_axis_name="c", subcore_axis_name="s")
_f32 = jnp.float32
_SC_PARAMS = pltpu.CompilerParams(use_tc_tiling_on_sc=False)


@functools.partial(
    pl.kernel,
    out_type=jax.ShapeDtypeStruct((NC, NP, D), _f32),
    mesh=_MESH,
    compiler_params=_SC_PARAMS,
    scratch_types=[
        pltpu.VMEM((NCH, CH), jnp.int32),
        pltpu.VMEM((NCH, CH), jnp.int32),
        pltpu.VMEM((CH, D), _f32),
        pltpu.VMEM_SHARED((NP, D), _f32),
    ],
)
def _sc_scatter(g_hbm, src_hbm, dst_hbm, zeros_hbm, out_hbm,
                sidx_v, didx_v, rows_v, acc_sp):
    c = lax.axis_index("c")
    s = lax.axis_index("s")
    wid = c * NS + s
    pltpu.sync_copy(zeros_hbm.at[pl.ds(s * RPS, RPS)],
                    acc_sp.at[pl.ds(s * RPS, RPS)])
    plsc.subcore_barrier()
    pltpu.sync_copy(src_hbm.at[wid], sidx_v)
    pltpu.sync_copy(dst_hbm.at[wid], didx_v)

    def chunk(j, carry):
        pltpu.sync_copy(g_hbm.at[sidx_v.at[j]], rows_v)
        pltpu.sync_copy(rows_v, acc_sp.at[didx_v.at[j]], add=True)
        return carry

    lax.fori_loop(0, NCH, chunk, 0)
    plsc.subcore_barrier()
    pltpu.sync_copy(acc_sp.at[pl.ds(s * RPS, RPS)],
                    out_hbm.at[c, pl.ds(s * RPS, RPS)])


@functools.partial(
    pl.kernel,
    out_type=jax.ShapeDtypeStruct((NC, NP, 8), _f32),
    mesh=_MESH,
    compiler_params=_SC_PARAMS,
    scratch_types=[
        pltpu.VMEM((NCH, CH), jnp.int32),
        pltpu.VMEM((CH, 8), _f32),
        pltpu.VMEM_SHARED((NP, 8), _f32),
    ],
)
def _sc_deg(dst_hbm, zeros_hbm, ones_hbm, out_hbm, didx_v, ones_v, acc_sp):
    c = lax.axis_index("c")
    s = lax.axis_index("s")
    wid = c * NS + s
    pltpu.sync_copy(zeros_hbm.at[pl.ds(s * RPS, RPS)],
                    acc_sp.at[pl.ds(s * RPS, RPS)])
    pltpu.sync_copy(ones_hbm, ones_v)
    plsc.subcore_barrier()
    pltpu.sync_copy(dst_hbm.at[wid], didx_v)

    def chunk(j, carry):
        pltpu.sync_copy(ones_v, acc_sp.at[didx_v.at[j]], add=True)
        return carry

    lax.fori_loop(0, NCH, chunk, 0)
    plsc.subcore_barrier()
    pltpu.sync_copy(acc_sp.at[pl.ds(s * RPS, RPS)],
                    out_hbm.at[c, pl.ds(s * RPS, RPS)])


@functools.partial(
    pl.kernel,
    out_type=jax.ShapeDtypeStruct((NP, D), _f32),
    mesh=_MESH,
    compiler_params=_SC_PARAMS,
    scratch_types=[
        pltpu.VMEM((GC, GB), jnp.int32),
        pltpu.VMEM((GB, D), _f32),
    ],
)
def _sc_permgather(xw_hbm, pidx_hbm, out_hbm, pidx_v, prow_v):
    c = lax.axis_index("c")
    s = lax.axis_index("s")
    wid = c * NS + s
    pltpu.sync_copy(pidx_hbm.at[wid], pidx_v)
    for j in range(GC):
        pltpu.sync_copy(xw_hbm.at[pidx_v.at[j]], prow_v)
        pltpu.sync_copy(prow_v, out_hbm.at[pl.ds(wid * (GC * GB) + j * GB, GB)])


def _dinv(degp):
    return lax.rsqrt(degp[0, :, 0:1] + degp[1, :, 0:1] + 1.0)


def _tc1_body(x_ref, w1_ref, wb_ref, degp_ref, g1_ref, xw_ref):
    dinv = _dinv(degp_ref[...])
    xx = x_ref[...]
    h2 = jnp.dot(xx, w1_ref[...], preferred_element_type=_f32)
    g1_ref[...] = h2 * dinv
    xw_ref[...] = jnp.dot(xx, wb_ref[...], preferred_element_type=_f32)


_tc1 = pl.pallas_call(
    _tc1_body,
    out_shape=[jax.ShapeDtypeStruct((NP, D), _f32),
               jax.ShapeDtypeStruct((NP, D), _f32)],
)


def _tc_layer_body(sp_ref, g_ref, degp_ref, b_ref, w_ref, out_ref):
    dinv = _dinv(degp_ref[...])
    sp = sp_ref[...]
    agg = dinv * (sp[0] + sp[1] + g_ref[...]) + b_ref[...]
    h = jnp.maximum(agg, 0.0)
    out_ref[...] = jnp.dot(h, w_ref[...], preferred_element_type=_f32) * dinv


_tc_layer = pl.pallas_call(
    _tc_layer_body,
    out_shape=jax.ShapeDtypeStruct((NP, D), _f32),
)


def _tc_final_body(sp_ref, g_ref, degp_ref, b_ref, xw_ref, xwp_ref, bb_ref,
                   lg_ref, ng_ref):
    dinv = _dinv(degp_ref[...])
    sp = sp_ref[...]
    emb = dinv * (sp[0] + sp[1] + g_ref[...]) + b_ref[...]
    lg_ref[...] = jnp.sum(xw_ref[...] * emb, axis=1, keepdims=True) + bb_ref[0, 0]
    ng_ref[...] = jnp.sum(xwp_ref[...] * emb, axis=1, keepdims=True) + bb_ref[0, 0]


_tc_final = pl.pallas_call(
    _tc_final_body,
    out_shape=[jax.ShapeDtypeStruct((NP, 1), _f32),
               jax.ShapeDtypeStruct((NP, 1), _f32)],
)


def kernel(x, edge_index, W1, b1, W2, b2, W3, b3, W4, b4, Wb, bb):
    src = edge_index[0].astype(jnp.int32)
    dst = edge_index[1].astype(jnp.int32)
    npad = EP - E
    ar = jnp.arange(npad, dtype=jnp.int32)
    src_p = jnp.concatenate([src, ar % N_NODES]).reshape(NW, NCH, CH)
    dst_p = jnp.concatenate([dst, N_NODES + ar % (NP - N_NODES)]).reshape(NW, NCH, CH)
    x_p = jnp.pad(x, ((0, NP - N_NODES), (0, 0)))
    zeros64 = jnp.zeros((NP, D), _f32)
    zeros8 = jnp.zeros((NP, 8), _f32)
    ones8 = jnp.ones((CH, 8), _f32)
    perm = jax.random.permutation(jax.random.key(1), N_NODES).astype(jnp.int32)
    perm_p = jnp.concatenate(
        [perm, jnp.arange(NP - N_NODES, dtype=jnp.int32) % N_NODES]
    ).reshape(NW, GC, GB)

    degp = _sc_deg(dst_p, zeros8, ones8)
    g1, xw = _tc1(x_p, W1, Wb[0], degp)
    xwp = _sc_permgather(xw, perm_p)
    S1 = _sc_scatter(g1, src_p, dst_p, zeros64)
    g2 = _tc_layer(S1, g1, degp, b1.reshape(1, D), W2)
    S2 = _sc_scatter(g2, src_p, dst_p, zeros64)
    g3 = _tc_layer(S2, g2, degp, b2.reshape(1, D), W3)
    S3 = _sc_scatter(g3, src_p, dst_p, zeros64)
    g4 = _tc_layer(S3, g3, degp, b3.reshape(1, D), W4)
    S4 = _sc_scatter(g4, src_p, dst_p, zeros64)
    lg, ng = _tc_final(S4, g4, degp, b4.reshape(1, D), xw, xwp,
                       bb.reshape(1, 1))
    return lg[:N_NODES, 0], ng[:N_NODES, 0]


# speedup vs baseline: 35.1977x; 1.6098x over previous
import functools

import jax
import jax.numpy as jnp
from jax import lax
from jax.experimental import pallas as pl
from jax.experimental.pallas import tpu as pltpu
from jax.experimental.pallas import tpu_sc as plsc

N_NODES = 10000
NP = 10240
E = 320000
EP = 323584
D = 64
NC, NS = 2, 16
NW = NC * NS
CH = 128
NCH = (EP // NW) // CH
RPS = NP // NS
GC, GB = 4, 80

_MESH = plsc.VectorSubcoreMesh(core_axis_name="c", subcore_axis_name="s")
_f32 = jnp.float32
_SC_PARAMS = pltpu.CompilerParams(use_tc_tiling_on_sc=False)


@functools.partial(
    pl.kernel,
    out_type=jax.ShapeDtypeStruct((NC, NP, D), _f32),
    mesh=_MESH,
    compiler_params=_SC_PARAMS,
    scratch_types=[
        pltpu.VMEM((NCH, CH), jnp.int32),
        pltpu.VMEM((NCH, CH), jnp.int32),
        pltpu.VMEM((3, CH, D), _f32),
        pltpu.VMEM_SHARED((NP, D), _f32),
        pltpu.SemaphoreType.DMA,
    ],
)
def _sc_scatter(g_hbm, src_hbm, dst_hbm, zeros_hbm, out_hbm,
                sidx_v, didx_v, rows_v, acc_sp, sem_g):
    c = lax.axis_index("c")
    s = lax.axis_index("s")
    wid = c * NS + s
    pltpu.sync_copy(zeros_hbm.at[pl.ds(s * RPS, RPS)],
                    acc_sp.at[pl.ds(s * RPS, RPS)])
    plsc.subcore_barrier()
    pltpu.sync_copy(src_hbm.at[wid], sidx_v)
    pltpu.sync_copy(dst_hbm.at[wid], didx_v)

    def start_gather(j):
        pltpu.async_copy(g_hbm.at[sidx_v.at[j]], rows_v.at[lax.rem(j, 3)],
                         sem_g)

    start_gather(0)
    start_gather(1)

    def chunk(j, carry):
        @pl.when(j + 2 < NCH)
        def _():
            start_gather(j + 2)
        pltpu.make_async_copy(g_hbm.at[sidx_v.at[j]],
                              rows_v.at[lax.rem(j, 3)], sem_g).wait()
        pltpu.sync_copy(rows_v.at[lax.rem(j, 3)],
                        acc_sp.at[didx_v.at[j]], add=True)
        return carry

    lax.fori_loop(0, NCH, chunk, 0)
    plsc.subcore_barrier()
    pltpu.sync_copy(acc_sp.at[pl.ds(s * RPS, RPS)],
                    out_hbm.at[c, pl.ds(s * RPS, RPS)])


@functools.partial(
    pl.kernel,
    out_type=jax.ShapeDtypeStruct((NC, NP, 8), _f32),
    mesh=_MESH,
    compiler_params=_SC_PARAMS,
    scratch_types=[
        pltpu.VMEM((NCH, CH), jnp.int32),
        pltpu.VMEM((CH, 8), _f32),
        pltpu.VMEM_SHARED((NP, 8), _f32),
    ],
)
def _sc_deg(dst_hbm, zeros_hbm, ones_hbm, out_hbm, didx_v, ones_v, acc_sp):
    c = lax.axis_index("c")
    s = lax.axis_index("s")
    wid = c * NS + s
    pltpu.sync_copy(zeros_hbm.at[pl.ds(s * RPS, RPS)],
                    acc_sp.at[pl.ds(s * RPS, RPS)])
    pltpu.sync_copy(ones_hbm, ones_v)
    plsc.subcore_barrier()
    pltpu.sync_copy(dst_hbm.at[wid], didx_v)

    def chunk(j, carry):
        pltpu.sync_copy(ones_v, acc_sp.at[didx_v.at[j]], add=True)
        return carry

    lax.fori_loop(0, NCH, chunk, 0)
    plsc.subcore_barrier()
    pltpu.sync_copy(acc_sp.at[pl.ds(s * RPS, RPS)],
                    out_hbm.at[c, pl.ds(s * RPS, RPS)])


@functools.partial(
    pl.kernel,
    out_type=jax.ShapeDtypeStruct((NP, D), _f32),
    mesh=_MESH,
    compiler_params=_SC_PARAMS,
    scratch_types=[
        pltpu.VMEM((GC, GB), jnp.int32),
        pltpu.VMEM((GB, D), _f32),
    ],
)
def _sc_permgather(xw_hbm, pidx_hbm, out_hbm, pidx_v, prow_v):
    c = lax.axis_index("c")
    s = lax.axis_index("s")
    wid = c * NS + s
    pltpu.sync_copy(pidx_hbm.at[wid], pidx_v)
    for j in range(GC):
        pltpu.sync_copy(xw_hbm.at[pidx_v.at[j]], prow_v)
        pltpu.sync_copy(prow_v, out_hbm.at[pl.ds(wid * (GC * GB) + j * GB, GB)])


def _dinv(degp):
    return lax.rsqrt(degp[0, :, 0:1] + degp[1, :, 0:1] + 1.0)


def _tc1_body(x_ref, w1_ref, wb_ref, degp_ref, g1_ref, xw_ref):
    dinv = _dinv(degp_ref[...])
    xx = x_ref[...]
    h2 = jnp.dot(xx, w1_ref[...], preferred_element_type=_f32)
    g1_ref[...] = h2 * dinv
    xw_ref[...] = jnp.dot(xx, wb_ref[...], preferred_element_type=_f32)


_tc1 = pl.pallas_call(
    _tc1_body,
    out_shape=[jax.ShapeDtypeStruct((NP, D), _f32),
               jax.ShapeDtypeStruct((NP, D), _f32)],
)


def _tc_layer_body(sp_ref, g_ref, degp_ref, b_ref, w_ref, out_ref):
    dinv = _dinv(degp_ref[...])
    sp = sp_ref[...]
    agg = dinv * (sp[0] + sp[1] + g_ref[...]) + b_ref[...]
    h = jnp.maximum(agg, 0.0)
    out_ref[...] = jnp.dot(h, w_ref[...], preferred_element_type=_f32) * dinv


_tc_layer = pl.pallas_call(
    _tc_layer_body,
    out_shape=jax.ShapeDtypeStruct((NP, D), _f32),
)


def _tc_final_body(sp_ref, g_ref, degp_ref, b_ref, xw_ref, xwp_ref, bb_ref,
                   lg_ref, ng_ref):
    dinv = _dinv(degp_ref[...])
    sp = sp_ref[...]
    emb = dinv * (sp[0] + sp[1] + g_ref[...]) + b_ref[...]
    lg_ref[...] = jnp.sum(xw_ref[...] * emb, axis=1, keepdims=True) + bb_ref[0, 0]
    ng_ref[...] = jnp.sum(xwp_ref[...] * emb, axis=1, keepdims=True) + bb_ref[0, 0]


_tc_final = pl.pallas_call(
    _tc_final_body,
    out_shape=[jax.ShapeDtypeStruct((NP, 1), _f32),
               jax.ShapeDtypeStruct((NP, 1), _f32)],
)


def kernel(x, edge_index, W1, b1, W2, b2, W3, b3, W4, b4, Wb, bb):
    src = edge_index[0].astype(jnp.int32)
    dst = edge_index[1].astype(jnp.int32)
    npad = EP - E
    ar = jnp.arange(npad, dtype=jnp.int32)
    src_p = jnp.concatenate([src, ar % N_NODES]).reshape(NW, NCH, CH)
    dst_p = jnp.concatenate([dst, N_NODES + ar % (NP - N_NODES)]).reshape(NW, NCH, CH)
    x_p = jnp.pad(x, ((0, NP - N_NODES), (0, 0)))
    zeros64 = jnp.zeros((NP, D), _f32)
    zeros8 = jnp.zeros((NP, 8), _f32)
    ones8 = jnp.ones((CH, 8), _f32)
    perm = jax.random.permutation(jax.random.key(1), N_NODES).astype(jnp.int32)
    perm_p = jnp.concatenate(
        [perm, jnp.arange(NP - N_NODES, dtype=jnp.int32) % N_NODES]
    ).reshape(NW, GC, GB)

    degp = _sc_deg(dst_p, zeros8, ones8)
    g1, xw = _tc1(x_p, W1, Wb[0], degp)
    xwp = _sc_permgather(xw, perm_p)
    S1 = _sc_scatter(g1, src_p, dst_p, zeros64)
    g2 = _tc_layer(S1, g1, degp, b1.reshape(1, D), W2)
    S2 = _sc_scatter(g2, src_p, dst_p, zeros64)
    g3 = _tc_layer(S2, g2, degp, b2.reshape(1, D), W3)
    S3 = _sc_scatter(g3, src_p, dst_p, zeros64)
    g4 = _tc_layer(S3, g3, degp, b3.reshape(1, D), W4)
    S4 = _sc_scatter(g4, src_p, dst_p, zeros64)
    lg, ng = _tc_final(S4, g4, degp, b4.reshape(1, D), xw, xwp,
                       bb.reshape(1, 1))
    return lg[:N_NODES, 0], ng[:N_NODES, 0]
